# baseline (speedup 1.0000x reference)
import functools

import jax
import jax.numpy as jnp
from jax import lax
from jax.experimental import pallas as pl
from jax.experimental.pallas import tpu as pltpu
from jax.experimental.pallas import tpu_sc as plsc

VOC = 1_000_000
EMB = 64
B = 16384
K = 20
R = 20
KR = K + R
PAIR = 2 * EMB
NPAIR = VOC // 2

NC = 2
NS = 16
NW = NC * NS
NLANE = 16
NV = EMB // NLANE

SLABW = 128
NSLAB = VOC // SLABW
SLAB_MAIN = NSLAB // NW
SLAB_EXTRA = NSLAB % NW
REMW = VOC - NSLAB * SLABW

NB = B // NW
SB = 8
NSUB = NB // SB
TASKS = SB * KR
GCHUNK = 64
NG = TASKS // GCHUNK
NGRP = TASKS // NLANE


@functools.partial(
    pl.kernel,
    out_type=(jax.ShapeDtypeStruct((NPAIR, PAIR), jnp.float32),
              jax.ShapeDtypeStruct((NPAIR, PAIR), jnp.float32)),
    mesh=plsc.VectorSubcoreMesh(core_axis_name="c", subcore_axis_name="s"),
    compiler_params=pltpu.CompilerParams(
        needs_layout_passes=False, use_tc_tiling_on_sc=True),
    scratch_types=[
        pltpu.VMEM((EMB, SLABW), jnp.float32),
        pltpu.VMEM((EMB, SLABW), jnp.float32),
        pltpu.VMEM((EMB, PAIR), jnp.float32),
        pltpu.VMEM((EMB, PAIR), jnp.float32),
        pltpu.SemaphoreType.DMA,
        pltpu.SemaphoreType.DMA,
        pltpu.SemaphoreType.DMA,
        pltpu.SemaphoreType.DMA,
    ],
)
def _sc_transpose(embT, lwT, emb_tail, lw_tail, embR, lwR,
                  sbuf0, sbuf1, obuf0, obuf1, si0, si1, so0, so1):
    wid = lax.axis_index("s") * NC + lax.axis_index("c")
    lane = lax.iota(jnp.int32, NLANE)

    def _in_copy(src, slab, sbuf, sem):
        v0 = pl.multiple_of(slab * SLABW, SLABW)
        return pltpu.make_async_copy(src.at[:, pl.ds(v0, SLABW)], sbuf, sem)

    def _out_copy(dst, slab, obuf, sem):
        r0 = pl.multiple_of(slab * EMB, EMB)
        return pltpu.make_async_copy(obuf, dst.at[pl.ds(r0, EMB)], sem)

    def _tpose(sbuf, obuf):
        @pl.loop(0, EMB)
        def _r(r):
            for j in range(NV):
                c0 = plsc.load_gather(
                    sbuf, [j * NLANE + lane,
                           2 * r + jnp.zeros((NLANE,), jnp.int32)])
                c1 = plsc.load_gather(
                    sbuf, [j * NLANE + lane,
                           2 * r + 1 + jnp.zeros((NLANE,), jnp.int32)])
                obuf[r, pl.ds(pl.multiple_of(j * NLANE, NLANE), NLANE)] = c0
                obuf[r, pl.ds(pl.multiple_of(EMB + j * NLANE, NLANE),
                              NLANE)] = c1

    def _sweep(src, dst, tail):
        _in_copy(src, wid, sbuf0, si0).start()

        @pl.loop(0, SLAB_MAIN // 2)
        def _h(h):
            s0 = (2 * h) * NW + wid

            @pl.when(2 * h + 1 < SLAB_MAIN)
            def _():
                _in_copy(src, s0 + NW, sbuf1, si1).start()

            _in_copy(src, s0, sbuf0, si0).wait()
            _tpose(sbuf0, obuf0)
            _out_copy(dst, s0, obuf0, so0).start()

            @pl.when((2 * h + 2 < SLAB_MAIN)
                     | ((wid < SLAB_EXTRA) & (2 * h + 2 < SLAB_MAIN + 4)))
            def _():
                nxt = jnp.where(2 * h + 2 < SLAB_MAIN,
                                (2 * h + 2) * NW + wid,
                                NSLAB - SLAB_EXTRA + wid)
                _in_copy(src, nxt, sbuf0, si0).start()

            _in_copy(src, s0 + NW, sbuf1, si1).wait()
            _tpose(sbuf1, obuf1)
            _out_copy(dst, s0 + NW, obuf1, so1).start()
            _out_copy(dst, s0, obuf0, so0).wait()
            _out_copy(dst, s0 + NW, obuf1, so1).wait()

        @pl.when(wid < SLAB_EXTRA)
        def _():
            s = NSLAB - SLAB_EXTRA + wid
            _in_copy(src, s, sbuf0, si0).wait()
            _tpose(sbuf0, obuf0)
            _out_copy(dst, s, obuf0, so0).start()
            _out_copy(dst, s, obuf0, so0).wait()

        @pl.when(wid == SLAB_EXTRA)
        def _():
            pltpu.sync_copy(tail, obuf0.at[pl.ds(0, REMW // 2)])
            r0 = pl.multiple_of(NSLAB * EMB, NLANE)
            pltpu.sync_copy(obuf0.at[pl.ds(0, REMW // 2)],
                            dst.at[pl.ds(r0, REMW // 2)])

    _sweep(embT, embR, emb_tail)
    _sweep(lwT, lwR, lw_tail)


@functools.partial(
    pl.kernel,
    out_type=jax.ShapeDtypeStruct((B * KR,), jnp.float32),
    mesh=plsc.VectorSubcoreMesh(core_axis_name="c", subcore_axis_name="s"),
    compiler_params=pltpu.CompilerParams(
        needs_layout_passes=False, use_tc_tiling_on_sc=True),
    scratch_types=[
        pltpu.VMEM((NB,), jnp.int32),
        pltpu.VMEM((NB * KR,), jnp.int32),
        pltpu.VMEM((NLANE,), jnp.int32),
        pltpu.VMEM((NLANE,), jnp.int32),
        pltpu.VMEM((TASKS,), jnp.int32),
        pltpu.VMEM((TASKS,), jnp.int32),
        pltpu.VMEM((SB, PAIR), jnp.float32),
        pltpu.VMEM((SB, PAIR), jnp.float32),
        pltpu.VMEM((TASKS, PAIR), jnp.float32),
        pltpu.VMEM((TASKS, PAIR), jnp.float32),
        pltpu.VMEM((TASKS * NLANE,), jnp.float32),
        pltpu.VMEM((TASKS,), jnp.float32),
        pltpu.SemaphoreType.DMA,
        pltpu.SemaphoreType.DMA,
    ],
)
def _sc_dots(center_hbm, cw_hbm, emb_hbm, lw_hbm, dots_hbm,
             cidx, widx, cp0, cp1, wp0, wp1, eb0, eb1, wb0, wb1,
             pbuf, dbuf, sem0, sem1):
    wid = lax.axis_index("s") * NC + lax.axis_index("c")
    b0 = pl.multiple_of(wid * NB, NB)
    t0 = pl.multiple_of(wid * (NB * KR), NB * KR)
    pltpu.sync_copy(center_hbm.at[pl.ds(b0, NB)], cidx)
    pltpu.sync_copy(cw_hbm.at[pl.ds(t0, NB * KR)], widx)

    lane = lax.iota(jnp.int32, NLANE)
    zero16 = jnp.zeros((NLANE,), jnp.int32)

    def _stage_idx(s, cp, wp):
        sb0 = pl.multiple_of(s * SB, SB)
        st0 = pl.multiple_of(s * TASKS, TASKS)
        cp[:] = lax.shift_right_logical(
            plsc.load_gather(cidx, [jnp.minimum(sb0 + lane, NB - 1)]), 1)

        @pl.loop(0, NGRP)
        def _i(i):
            o = pl.multiple_of(i * NLANE, NLANE)
            wp[pl.ds(o, NLANE)] = lax.shift_right_logical(
                widx[pl.ds(st0 + o, NLANE)], 1)

    def _copies(cp, wp, eb, wb, sem):
        yield pltpu.make_async_copy(emb_hbm.at[cp.at[pl.ds(0, SB)]], eb, sem)
        for q in range(NG):
            yield pltpu.make_async_copy(
                lw_hbm.at[wp.at[pl.ds(q * GCHUNK, GCHUNK)]],
                wb.at[pl.ds(q * GCHUNK, GCHUNK)], sem)

    def _issue(s, cp, wp, eb, wb, sem):
        _stage_idx(s, cp, wp)
        for c in _copies(cp, wp, eb, wb, sem):
            c.start()

    def _wait(cp, wp, eb, wb, sem):
        for c in _copies(cp, wp, eb, wb, sem):
            c.wait()

    def _compute(s, eb, wb):
        sb0 = pl.multiple_of(s * SB, SB)
        st0 = pl.multiple_of(s * TASKS, TASKS)

        @pl.loop(0, SB)
        def _per_b(b):
            ch = plsc.load_gather(cidx, [zero16 + (sb0 + b)]) & 1
            codd = ch == 1
            e = [jnp.where(codd,
                           eb[b, pl.ds(EMB + j * NLANE, NLANE)],
                           eb[b, pl.ds(j * NLANE, NLANE)])
                 for j in range(NV)]

            @pl.loop(0, KR)
            def _per_k(k):
                t = b * KR + k
                wh = plsc.load_gather(widx, [zero16 + (st0 + t)]) & 1
                wodd = wh == 1
                p = jnp.where(wodd,
                              wb[t, pl.ds(EMB, NLANE)],
                              wb[t, pl.ds(0, NLANE)]) * e[0]
                for j in range(1, NV):
                    p = p + jnp.where(
                        wodd,
                        wb[t, pl.ds(EMB + j * NLANE, NLANE)],
                        wb[t, pl.ds(j * NLANE, NLANE)]) * e[j]
                pbuf[pl.ds(pl.multiple_of(t * NLANE, NLANE), NLANE)] = p

        @pl.loop(0, NGRP)
        def _per_g(g):
            base = g * (NLANE * NLANE) + lane * NLANE
            acc = plsc.load_gather(pbuf, [base])
            for j in range(1, NLANE):
                acc = acc + plsc.load_gather(pbuf, [base + j])
            dbuf[pl.ds(pl.multiple_of(g * NLANE, NLANE), NLANE)] = acc

        pltpu.sync_copy(dbuf, dots_hbm.at[pl.ds(t0 + st0, TASKS)])

    _issue(0, cp0, wp0, eb0, wb0, sem0)

    @pl.loop(0, NSUB // 2)
    def _pair(h):
        s0 = h * 2
        _issue(s0 + 1, cp1, wp1, eb1, wb1, sem1)
        _wait(cp0, wp0, eb0, wb0, sem0)
        _compute(s0, eb0, wb0)

        @pl.when(h < NSUB // 2 - 1)
        def _():
            _issue(s0 + 2, cp0, wp0, eb0, wb0, sem0)

        _wait(cp1, wp1, eb1, wb1, sem1)
        _compute(s0 + 1, eb1, wb1)


def _tc_loss_body(d_ref, o_ref):
    d = d_ref[...]
    col = lax.broadcasted_iota(jnp.int32, (B, KR), 1)
    act = jax.nn.sigmoid(d)
    pos = -jnp.log(act)
    neg = -jnp.log(1.0 - act + 1e-3)
    is_pos = col < K
    s_pos = jnp.sum(jnp.where(is_pos, pos, 0.0))
    s_neg = jnp.sum(jnp.where(is_pos, 0.0, neg))
    o_ref[0, 0] = s_pos / (B * K) + s_neg / (B * R)


_tc_loss = pl.pallas_call(
    _tc_loss_body,
    out_shape=jax.ShapeDtypeStruct((1, 1), jnp.float32),
    out_specs=pl.BlockSpec(memory_space=pltpu.SMEM),
)


def kernel(center, context, rand, embeddings, linear_w):
    center = center.astype(jnp.int32)
    cw = jnp.concatenate([context, rand], axis=1).astype(jnp.int32)
    emb_tail = embeddings[NSLAB * SLABW:].reshape(REMW // 2, PAIR)
    lw_tail = linear_w[NSLAB * SLABW:].reshape(REMW // 2, PAIR)
    embR, lwR = _sc_transpose(embeddings.T, linear_w.T, emb_tail, lw_tail)
    dots = _sc_dots(center, cw.reshape(-1), embR, lwR)
    loss = _tc_loss(dots.reshape(B, KR))
    return loss[0, 0]
